# baseline (speedup 1.0000x reference)
import jax
import jax.numpy as jnp
from jax import lax
from jax.experimental import pallas as pl
from jax.experimental.pallas import tpu as pltpu
from jax.experimental.pallas import tpu_sc as plsc

NC, NS = 2, 16
NW = NC * NS
B, S, H = 4, 8192, 1024
SPW = S // NW
CS = 8
NCHUNK = SPW // CS
NSTEP = NCHUNK * B
NBUF = 12
DEPTH = 8
UNROLL = 4


def _body(ids_hbm, tok_hbm, pos_hbm, out_hbm, *scr):
    idx_v = scr[0]
    tok_bufs = scr[1:1 + NBUF]
    pos_bufs = scr[1 + NBUF:3 + NBUF]
    gsem = scr[3 + NBUF:3 + 2 * NBUF]
    ssem = scr[3 + 2 * NBUF:3 + 3 * NBUF]
    psem = scr[3 + 3 * NBUF:5 + 3 * NBUF]

    wid = lax.axis_index("s") * NC + lax.axis_index("c")
    s_base = wid * SPW

    pltpu.sync_copy(ids_hbm.at[wid], idx_v)

    def issue_pos(c):
        return pltpu.async_copy(
            pos_hbm.at[pl.ds(s_base + c * CS, CS)], pos_bufs[c % 2],
            psem[c % 2])

    def issue_gather(i):
        c, b = i // B, i % B
        return pltpu.async_copy(
            tok_hbm.at[idx_v.at[b, pl.ds(c * CS, CS)]], tok_bufs[i % NBUF],
            gsem[i % NBUF])

    def issue_store(i):
        c, b = i // B, i % B
        return pltpu.async_copy(
            tok_bufs[i % NBUF],
            out_hbm.at[pl.ds(b * S + (s_base + c * CS), CS)], ssem[i % NBUF])

    pos_d = {0: issue_pos(0), 1: issue_pos(1)}
    gat_d = {i: issue_gather(i) for i in range(DEPTH)}
    sto_d = {}

    for i in range(NSTEP):
        c, b = i // B, i % B
        tok_v = tok_bufs[i % NBUF]
        pos_v = pos_bufs[c % 2]

        gat_d.pop(i).wait()
        if b == 0:
            pos_d.pop(c).wait()

        if i + DEPTH < NSTEP:
            j = i + DEPTH
            if j - NBUF in sto_d:
                sto_d.pop(j - NBUF).wait()
            gat_d[j] = issue_gather(j)

        @plsc.parallel_loop(0, CS * (H // 16), unroll=UNROLL)
        def _add(g):
            r = g >> 6
            sl = pl.ds((g & (H // 16 - 1)) * 16, 16)
            plsc.addupdate(tok_v.at[r, sl], pos_v[r, sl])

        sto_d[i] = issue_store(i)
        if b == B - 1 and c + 2 < NCHUNK:
            pos_d[c + 2] = issue_pos(c + 2)

    for i in sorted(sto_d):
        sto_d.pop(i).wait()


@jax.jit
def _embed(ids, tok_table, pos_used):
    mesh = plsc.VectorSubcoreMesh(core_axis_name="c", subcore_axis_name="s")
    f = pl.kernel(
        _body,
        out_type=jax.ShapeDtypeStruct((B * S, H), jnp.float32),
        mesh=mesh,
        scratch_types=(
            [pltpu.VMEM((B, SPW), jnp.int32)]
            + [pltpu.VMEM((CS, H), jnp.float32) for _ in range(NBUF + 2)]
            + [pltpu.SemaphoreType.DMA for _ in range(NBUF * 2 + 2)]
        ),
    )
    return f(ids, tok_table, pos_used)


def kernel(input_ids, past_seq_len, tok_table, pos_table):
    b, s = input_ids.shape
    _, h = tok_table.shape
    pos_used = lax.dynamic_slice_in_dim(pos_table, past_seq_len, s, axis=0)
    ids = (input_ids.astype(jnp.int32)
           .reshape(b, NW, s // NW)
           .transpose(1, 0, 2))
    out = _embed(ids, tok_table, pos_used)
    return out.reshape(b, s, h)


# speedup vs baseline: 1.0139x; 1.0139x over previous
import jax
import jax.numpy as jnp
from jax import lax
from jax.experimental import pallas as pl
from jax.experimental.pallas import tpu as pltpu
from jax.experimental.pallas import tpu_sc as plsc

NC, NS = 2, 16
NW = NC * NS
B, S, H = 4, 8192, 1024
SPW = S // NW
CS = 16
NCHUNK = SPW // CS
NSTEP = NCHUNK * B
NBUF = 5
DEPTH = 3
UNROLL = 4


def _body(ids_hbm, tok_hbm, pos_hbm, out_hbm,
          idx_v, t0, t1, t2, t3, t4, p0, p1,
          g0, g1, g2, g3, g4, s0, s1, s2, s3, s4, q0, q1):
    tok_bufs = (t0, t1, t2, t3, t4)
    pos_bufs = (p0, p1)
    gsem = (g0, g1, g2, g3, g4)
    ssem = (s0, s1, s2, s3, s4)
    psem = (q0, q1)

    wid = lax.axis_index("s") * NC + lax.axis_index("c")
    s_base = wid * SPW

    pltpu.sync_copy(ids_hbm.at[wid], idx_v)

    def issue_pos(c):
        return pltpu.async_copy(
            pos_hbm.at[pl.ds(s_base + c * CS, CS)], pos_bufs[c % 2],
            psem[c % 2])

    def issue_gather(i):
        c, b = i // B, i % B
        return pltpu.async_copy(
            tok_hbm.at[idx_v.at[b, pl.ds(c * CS, CS)]], tok_bufs[i % NBUF],
            gsem[i % NBUF])

    def issue_store(i):
        c, b = i // B, i % B
        return pltpu.async_copy(
            tok_bufs[i % NBUF],
            out_hbm.at[pl.ds(b * S + (s_base + c * CS), CS)], ssem[i % NBUF])

    pos_d = {0: issue_pos(0), 1: issue_pos(1)}
    gat_d = {i: issue_gather(i) for i in range(DEPTH)}
    sto_d = {}

    for i in range(NSTEP):
        c, b = i // B, i % B
        tok_v = tok_bufs[i % NBUF]
        pos_v = pos_bufs[c % 2]

        gat_d.pop(i).wait()
        if b == 0:
            pos_d.pop(c).wait()

        if i + DEPTH < NSTEP:
            j = i + DEPTH
            if j - NBUF in sto_d:
                sto_d.pop(j - NBUF).wait()
            gat_d[j] = issue_gather(j)

        @plsc.parallel_loop(0, CS * (H // 16), unroll=UNROLL)
        def _add(g):
            r = g >> 6
            sl = pl.ds((g & (H // 16 - 1)) * 16, 16)
            plsc.addupdate(tok_v.at[r, sl], pos_v[r, sl])

        sto_d[i] = issue_store(i)
        if b == B - 1 and c + 2 < NCHUNK:
            pos_d[c + 2] = issue_pos(c + 2)

    for i in sorted(sto_d):
        sto_d.pop(i).wait()


@jax.jit
def _embed(ids, tok_table, pos_used):
    mesh = plsc.VectorSubcoreMesh(core_axis_name="c", subcore_axis_name="s")
    f = pl.kernel(
        _body,
        out_type=jax.ShapeDtypeStruct((B * S, H), jnp.float32),
        mesh=mesh,
        scratch_types=(
            [pltpu.VMEM((B, SPW), jnp.int32)]
            + [pltpu.VMEM((CS, H), jnp.float32) for _ in range(NBUF + 2)]
            + [pltpu.SemaphoreType.DMA for _ in range(NBUF * 2 + 2)]
        ),
    )
    return f(ids, tok_table, pos_used)


def kernel(input_ids, past_seq_len, tok_table, pos_table):
    b, s = input_ids.shape
    _, h = tok_table.shape
    pos_used = lax.dynamic_slice_in_dim(pos_table, past_seq_len, s, axis=0)
    ids = (input_ids.astype(jnp.int32)
           .reshape(b, NW, s // NW)
           .transpose(1, 0, 2))
    out = _embed(ids, tok_table, pos_used)
    return out.reshape(b, s, h)
